# Initial kernel scaffold; baseline (speedup 1.0000x reference)
import jax
import jax.numpy as jnp
from jax.experimental import pallas as pl


def kernel(data, l1_word_emb, cg_emb_0, cg_emb_1, cg_emb_2, spell_0, spell_1, spell_2):
    raise NotImplementedError("write your pallas kernel here")


# speedup vs baseline: 19.6080x; 19.6080x over previous
import functools

import jax
import jax.numpy as jnp
from jax import lax
from jax.experimental import pallas as pl
from jax.experimental.pallas import tpu as pltpu
from jax.experimental.pallas import tpu_sc as plsc

NC, NS, LANES = 2, 16, 16
NW = NC * NS
VOCAB = 100000
D = 64
SPELL = 8
N_ORD = 3

C1 = 80
G1 = VOCAB // C1
SP1 = C1 * SPELL
GATH = 128
NGATH = SP1 // GATH

C2 = 128

_mesh = dict(core_axis_name="c", subcore_axis_name="s")


def _worker_id():
    return lax.axis_index("s") * NC + lax.axis_index("c")


def _phase1(word_hbm, cg0, cg1, cg2, sp0, sp1, sp2, fused, spb, cgb, abuf, sem):
    wid = _worker_id()
    cgs = (cg0, cg1, cg2)
    sps = (sp0, sp1, sp2)
    n_extra = G1 % NW
    n_chunks = G1 // NW + jnp.where(wid < n_extra, 1, 0)

    def chunk_body(k, _):
        g = wid + k * NW
        base = g * C1
        pltpu.sync_copy(word_hbm.at[pl.ds(base, C1)], abuf)
        for cg, sp in zip(cgs, sps):
            pltpu.sync_copy(sp.at[g], spb)
            descs = [
                pltpu.async_copy(
                    cg.at[spb.at[pl.ds(j * GATH, GATH)]],
                    cgb.at[pl.ds(j * GATH, GATH)],
                    sem,
                )
                for j in range(NGATH)
            ]

            for d_ in descs:
                d_.wait()

            def pair_body(tp, carry):
                spv = spb[pl.ds(pl.multiple_of(tp * LANES, LANES), LANES)]
                b = jnp.where(spv != 0, 1.0, 0.0)
                for st in (1, 2, 4):
                    perm = jnp.arange(LANES, dtype=jnp.int32) ^ st
                    b = b + b.at[perm].get(mode="promise_in_bounds")
                rec = 1.0 / (b + 0.01)
                for tt in range(2):
                    t = tp * 2 + tt
                    rv = jnp.full((LANES,), rec[tt * SPELL], jnp.float32)
                    r0 = t * SPELL
                    s_regs = [cgb[r0, pl.ds(q * LANES, LANES)] for q in range(4)]
                    for s in range(1, SPELL):
                        for q in range(4):
                            s_regs[q] = s_regs[q] + cgb[r0 + s, pl.ds(q * LANES, LANES)]
                    for q in range(4):
                        sl = pl.ds(q * LANES, LANES)
                        abuf[t, sl] = abuf[t, sl] + s_regs[q] * rv
                return carry

            lax.fori_loop(0, C1 // 2, pair_body, 0)
        pltpu.sync_copy(abuf, fused.at[pl.ds(base, C1)])
        return _

    lax.fori_loop(0, n_chunks, chunk_body, 0)


def _phase2(fused, data2d, out, idxb, rows, sem):
    wid = _worker_id()
    n_chunks = data2d.shape[0] // NW

    def body(k, _):
        c = wid * n_chunks + k
        pltpu.sync_copy(data2d.at[c], idxb)
        pltpu.async_copy(fused.at[idxb], rows, sem).wait()
        pltpu.sync_copy(rows, out.at[pl.ds(c * C2, C2)])
        return _

    lax.fori_loop(0, n_chunks, body, 0)


def kernel(data, l1_word_emb, cg_emb_0, cg_emb_1, cg_emb_2, spell_0, spell_1, spell_2):
    n_tok = data.size
    data_flat = data.reshape(n_tok // C2, C2).astype(jnp.int32)
    sps = [s.astype(jnp.int32).reshape(G1, SP1) for s in (spell_0, spell_1, spell_2)]

    params = pltpu.CompilerParams(use_tc_tiling_on_sc=False)
    p1 = functools.partial(
        pl.kernel,
        out_type=jax.ShapeDtypeStruct((VOCAB, D), jnp.float32),
        mesh=plsc.VectorSubcoreMesh(**_mesh),
        compiler_params=params,
        scratch_types=[
            pltpu.VMEM((SP1,), jnp.int32),
            pltpu.VMEM((SP1, D), jnp.float32),
            pltpu.VMEM((C1, D), jnp.float32),
            pltpu.SemaphoreType.DMA,
        ],
    )(_phase1)
    fused = p1(l1_word_emb, cg_emb_0, cg_emb_1, cg_emb_2, *sps)

    p2 = functools.partial(
        pl.kernel,
        out_type=jax.ShapeDtypeStruct((n_tok, D), jnp.float32),
        mesh=plsc.VectorSubcoreMesh(**_mesh),
        compiler_params=params,
        scratch_types=[
            pltpu.VMEM((C2,), jnp.int32),
            pltpu.VMEM((C2, D), jnp.float32),
            pltpu.SemaphoreType.DMA,
        ],
    )(_phase2)
    out = p2(fused, data_flat)
    return out.reshape(data.shape + (D,))


# speedup vs baseline: 28.0042x; 1.4282x over previous
import jax
import jax.numpy as jnp
from jax import lax
from jax.experimental import pallas as pl
from jax.experimental.pallas import tpu as pltpu
from jax.experimental.pallas import tpu_sc as plsc

NC, NS, LANES = 2, 16, 16
NW = NC * NS
VOCAB = 100000
CGV = 50000
D = 64
SPELL = 8
N_ORD = 3
NQ = D // LANES
NH = D // 32

C1 = 80
G1 = VOCAB // C1
SP1 = C1 * SPELL
GATH = 128
NG1 = SP1 // GATH

NA, NB = 40, 38
XA = (G1 - NW * NB) // 2

C2 = 640
NG2 = C2 // GATH


def _worker_id():
    return lax.axis_index("s") * NC + lax.axis_index("c")


def _unit(cgb, sp, abuf, pad_id):
    def pair_body(tp, carry):
        spv = sp[pl.ds(pl.multiple_of(tp * LANES, LANES), LANES)]
        b = jnp.where(spv != pad_id, 1.0, 0.0)
        for st in (1, 2, 4):
            perm = jnp.arange(LANES, dtype=jnp.int32) ^ st
            b = b + b.at[perm].get(mode="promise_in_bounds")
        rec = 1.0 / (b + 0.01)
        for tt in range(2):
            t = tp * 2 + tt
            rv = jnp.full((LANES,), rec[tt * SPELL], jnp.float32)
            r0 = t * SPELL
            s_regs = [cgb[r0, pl.ds(h * 32, 32)] for h in range(NH)]
            for s in range(1, SPELL):
                for h in range(NH):
                    s_regs[h] = s_regs[h] + cgb[r0 + s, pl.ds(h * 32, 32)]
            for h in range(NH):
                lo, hi = lax.split(s_regs[h], [LANES, LANES])
                for q, part in ((2 * h, lo), (2 * h + 1, hi)):
                    sl = pl.ds(q * LANES, LANES)
                    abuf[t, sl] = abuf[t, sl] + part.astype(jnp.float32) * rv
        return carry

    lax.fori_loop(0, C1 // 2, pair_body, 0)


def _phase1(word_hbm, cg_all, sp_all, fused,
            spa0, spa1, spa2, spb0, spb1, spb2, cgbuf0, cgbuf1, abufA, abufB,
            gsem0, gsem1, ssemA, ssemB, vsemA, vsemB, wsemA, wsemB):
    wid = _worker_id()
    spa = (spa0, spa1, spa2)
    spb = (spb0, spb1, spb2)
    n_chunks = jnp.where(wid < XA, NA, NB)
    start = jnp.where(wid < XA, NA * wid, XA * NA + NB * (wid - XA))
    n_pair = n_chunks // 2

    def fire_spells(g, bufs, sem):
        for o in range(N_ORD):
            pltpu.async_copy(sp_all.at[o, g], bufs[o], sem)

    def drain_spells(bufs, sem):
        for o in range(N_ORD):
            pltpu.make_async_copy(sp_all.at[0, 0], bufs[o], sem).wait()

    def fire_word(g, abuf, sem):
        pltpu.async_copy(word_hbm.at[pl.ds(g * C1, C1)], abuf, sem)

    def drain_word(abuf, sem):
        pltpu.make_async_copy(word_hbm.at[pl.ds(0, C1)], abuf, sem).wait()

    def fire_cg(sp, cgb, sem):
        for j in range(NG1):
            pltpu.async_copy(
                cg_all.at[sp.at[pl.ds(j * GATH, GATH)]],
                cgb.at[pl.ds(j * GATH, GATH)],
                sem,
            )

    def drain_cg(cgb, sem):
        pltpu.make_async_copy(cg_all.at[pl.ds(0, SP1)], cgb, sem).wait()

    def fire_wb(abuf, g, sem):
        pltpu.async_copy(abuf, fused.at[pl.ds(g * C1, C1)], sem)

    def drain_wb(abuf, sem):
        pltpu.make_async_copy(abuf, fused.at[pl.ds(0, C1)], sem).wait()

    fire_spells(start, spa, ssemA)
    fire_spells(start + 1, spb, ssemB)
    fire_word(start, abufA, vsemA)
    drain_spells(spa, ssemA)
    fire_cg(spa0, cgbuf0, gsem0)

    PAD = [o * CGV for o in range(N_ORD)]

    def pair(k, carry):
        a = start + 2 * k
        b = a + 1
        more = k < n_pair - 1
        drain_word(abufA, vsemA)
        fire_cg(spa1, cgbuf1, gsem1)
        drain_cg(cgbuf0, gsem0)
        _unit(cgbuf0, spa0, abufA, PAD[0])
        fire_cg(spa2, cgbuf0, gsem0)
        drain_spells(spb, ssemB)

        @pl.when(k > 0)
        def _():
            drain_wb(abufB, wsemB)

        fire_word(b, abufB, vsemB)
        drain_cg(cgbuf1, gsem1)
        _unit(cgbuf1, spa1, abufA, PAD[1])
        fire_cg(spb0, cgbuf1, gsem1)
        drain_cg(cgbuf0, gsem0)
        _unit(cgbuf0, spa2, abufA, PAD[2])
        fire_wb(abufA, a, wsemA)

        @pl.when(more)
        def _():
            fire_spells(a + 2, spa, ssemA)

        fire_cg(spb1, cgbuf0, gsem0)
        drain_word(abufB, vsemB)
        drain_cg(cgbuf1, gsem1)
        _unit(cgbuf1, spb0, abufB, PAD[0])
        drain_wb(abufA, wsemA)

        @pl.when(more)
        def _():
            fire_word(a + 2, abufA, vsemA)

        fire_cg(spb2, cgbuf1, gsem1)
        drain_cg(cgbuf0, gsem0)
        _unit(cgbuf0, spb1, abufB, PAD[1])
        @pl.when(more)
        def _():
            drain_spells(spa, ssemA)
            fire_cg(spa0, cgbuf0, gsem0)

        drain_cg(cgbuf1, gsem1)
        _unit(cgbuf1, spb2, abufB, PAD[2])
        fire_wb(abufB, b, wsemB)

        @pl.when(more)
        def _():
            fire_spells(b + 2, spb, ssemB)

        return carry

    lax.fori_loop(0, n_pair, pair, 0)
    drain_wb(abufB, wsemB)


def _phase2(fused, data2d, out, ib0, ib1, rows0, rows1,
            isem0, isem1, gsem0, gsem1, wsem0, wsem1):
    wid = _worker_id()
    nch = data2d.shape[0] // NW
    start = wid * nch
    n_pair = nch // 2

    def fire_idx(c, ib, sem):
        pltpu.async_copy(data2d.at[c], ib, sem)

    def drain_idx(ib, sem):
        pltpu.make_async_copy(data2d.at[0], ib, sem).wait()

    def fire_g(ib, rows, sem):
        for j in range(NG2):
            pltpu.async_copy(
                fused.at[ib.at[pl.ds(j * GATH, GATH)]],
                rows.at[pl.ds(j * GATH, GATH)],
                sem,
            )

    def drain_g(rows, sem):
        pltpu.make_async_copy(fused.at[pl.ds(0, C2)], rows, sem).wait()

    def fire_w(rows, c, sem):
        pltpu.async_copy(rows, out.at[pl.ds(c * C2, C2)], sem)

    def drain_w(rows, sem):
        pltpu.make_async_copy(rows, out.at[pl.ds(0, C2)], sem).wait()

    fire_idx(start, ib0, isem0)
    fire_idx(start + 1, ib1, isem1)
    drain_idx(ib0, isem0)
    fire_g(ib0, rows0, gsem0)

    def pair(k, carry):
        a = start + 2 * k
        b = a + 1
        more = k < n_pair - 1
        drain_idx(ib1, isem1)
        fire_g(ib1, rows1, gsem1)
        drain_g(rows0, gsem0)
        fire_w(rows0, a, wsem0)

        @pl.when(more)
        def _():
            fire_idx(a + 2, ib0, isem0)

        drain_w(rows0, wsem0)

        @pl.when(more)
        def _():
            drain_idx(ib0, isem0)
            fire_g(ib0, rows0, gsem0)

        drain_g(rows1, gsem1)
        fire_w(rows1, b, wsem1)

        @pl.when(more)
        def _():
            fire_idx(b + 2, ib1, isem1)

        drain_w(rows1, wsem1)
        return carry

    lax.fori_loop(0, n_pair, pair, 0)


def kernel(data, l1_word_emb, cg_emb_0, cg_emb_1, cg_emb_2, spell_0, spell_1, spell_2):
    n_tok = data.size
    data2d = data.reshape(n_tok // C2, C2).astype(jnp.int32)
    cg_all = jnp.concatenate([cg_emb_0, cg_emb_1, cg_emb_2], axis=0).astype(jnp.bfloat16)
    sp_all = jnp.stack(
        [spell_0.astype(jnp.int32), spell_1.astype(jnp.int32) + CGV,
         spell_2.astype(jnp.int32) + 2 * CGV]
    ).reshape(N_ORD, G1, SP1)

    params = pltpu.CompilerParams(use_tc_tiling_on_sc=False)
    p1 = pl.kernel(
        _phase1,
        out_type=jax.ShapeDtypeStruct((VOCAB, D), jnp.float32),
        mesh=plsc.VectorSubcoreMesh(core_axis_name="c", subcore_axis_name="s"),
        compiler_params=params,
        scratch_types=(
            [pltpu.VMEM((SP1,), jnp.int32) for _ in range(6)]
            + [pltpu.VMEM((SP1, D), jnp.bfloat16) for _ in range(2)]
            + [pltpu.VMEM((C1, D), jnp.float32) for _ in range(2)]
            + [pltpu.SemaphoreType.DMA] * 8
        ),
    )
    fused = p1(l1_word_emb, cg_all, sp_all)

    p2 = pl.kernel(
        _phase2,
        out_type=jax.ShapeDtypeStruct((n_tok, D), jnp.float32),
        mesh=plsc.VectorSubcoreMesh(core_axis_name="c", subcore_axis_name="s"),
        compiler_params=params,
        scratch_types=(
            [pltpu.VMEM((C2,), jnp.int32) for _ in range(2)]
            + [pltpu.VMEM((C2, D), jnp.float32) for _ in range(2)]
            + [pltpu.SemaphoreType.DMA] * 6
        ),
    )
    out = p2(fused, data2d)
    return out.reshape(data.shape + (D,))
